# baseline (speedup 1.0000x reference)
import functools
import math

import jax
import jax.numpy as jnp
from jax import lax
from jax.experimental import pallas as pl
from jax.experimental.pallas import tpu as pltpu
from jax.experimental.pallas import tpu_sc as plsc

VOCAB = 1000000
D = 64
ROWS = 4096
COLS = 200
B = ROWS * COLS
SCALE = math.sqrt(D)

NC = 2
NS = 16
NW = NC * NS
BPW = B // NW
CH = 400
NCH = BPW // CH
LANES = 16


def _emb_body(x_hbm, lut_hbm, out_hbm,
              idx_all, g0, g1, w0, w1,
              sg0, sg1, sw0, sw1):
    wid = lax.axis_index("s") * NC + lax.axis_index("c")
    base = wid * BPW
    gbuf = (g0, g1)
    wbuf = (w0, w1)
    sg = (sg0, sg1)
    sw = (sw0, sw1)

    pltpu.sync_copy(x_hbm.at[pl.ds(base, BPW)], idx_all)

    def start_gather(i, b):
        pltpu.async_copy(
            lut_hbm.at[idx_all.at[pl.ds(i * CH, CH)]], gbuf[b], sg[b])

    def start_write(i, b):
        pltpu.async_copy(
            wbuf[b], out_hbm.at[pl.ds(base + i * CH, CH)], sw[b])

    start_gather(0, 0)
    start_gather(1, 1)

    def step(i, b):
        pltpu.make_async_copy(
            lut_hbm.at[idx_all.at[pl.ds(0, CH)]], gbuf[b], sg[b]).wait()

        @pl.when(i >= 2)
        def _():
            pltpu.make_async_copy(
                wbuf[b], out_hbm.at[pl.ds(0, CH)], sw[b]).wait()

        def scale_row(r, c2):
            for j in range(D // LANES):
                sl = pl.ds(j * LANES, LANES)
                wbuf[b][r, sl] = gbuf[b][r, sl] * SCALE
            return c2

        lax.fori_loop(0, CH, scale_row, 0, unroll=4)
        start_write(i, b)

        @pl.when(i + 2 < NCH)
        def _():
            start_gather(i + 2, b)

    def pair(p, carry):
        step(2 * p, 0)
        step(2 * p + 1, 1)
        return carry

    lax.fori_loop(0, NCH // 2, pair, 0)

    pltpu.make_async_copy(w0, out_hbm.at[pl.ds(0, CH)], sw0).wait()
    pltpu.make_async_copy(w1, out_hbm.at[pl.ds(0, CH)], sw1).wait()


@jax.jit
def _emb(x_flat, lut):
    mesh = plsc.VectorSubcoreMesh(core_axis_name="c", subcore_axis_name="s")
    kern = functools.partial(
        pl.kernel,
        mesh=mesh,
        out_type=jax.ShapeDtypeStruct((B, D), jnp.float32),
        scratch_types=[
            pltpu.VMEM((BPW,), jnp.int32),
            pltpu.VMEM((CH, D), jnp.float32),
            pltpu.VMEM((CH, D), jnp.float32),
            pltpu.VMEM((CH, D), jnp.float32),
            pltpu.VMEM((CH, D), jnp.float32),
            pltpu.SemaphoreType.DMA,
            pltpu.SemaphoreType.DMA,
            pltpu.SemaphoreType.DMA,
            pltpu.SemaphoreType.DMA,
        ],
        compiler_params=pltpu.CompilerParams(use_tc_tiling_on_sc=False),
    )(_emb_body)
    return kern(x_flat, lut)


def kernel(x, lut):
    out = _emb(x.reshape(B).astype(jnp.int32), lut)
    return out.reshape(ROWS, COLS, D)
